# baseline (speedup 1.0000x reference)
import functools

import jax
import jax.numpy as jnp
from jax import lax
from jax.experimental import pallas as pl
from jax.experimental.pallas import tpu as pltpu
from jax.experimental.pallas import tpu_sc as plsc

N = 50000
E = 800000
G = 64
HID = 64

NC = 2
NS = 16
NW = NC * NS

CHUNK = 128
NCHUNK = E // CHUNK
P1_ITERS = 196
P2_ITERS = 392
E_PAD = 6272 * CHUNK

HALF = N // 2
AGG_ROWS = 25088

BLK = 2000
NBLK = N // BLK

_sc_params = pltpu.CompilerParams(use_tc_tiling_on_sc=False)


def _sc_mesh():
    return plsc.VectorSubcoreMesh(core_axis_name="c", subcore_axis_name="s")


def _permute16(x, idx):
    dn = lax.GatherDimensionNumbers(
        offset_dims=(), collapsed_slice_dims=(0,), start_index_map=(0,))
    return lax.gather(x, idx[:, None], dn, slice_sizes=(1,),
                      mode=lax.GatherScatterMode.PROMISE_IN_BOUNDS)


def _edge_logits_body(dst_h, src_h, q_h, k_h, e_h, sp_h, dstb, srcb, qb, kb,
                      eb, zvec, s_sh, semr0, semr1):
    c = lax.axis_index("c")
    s = lax.axis_index("s")
    wid = s * NC + c
    lane = lax.iota(jnp.int32, 16)
    semr = (semr0, semr1)

    def zb(i, carry):
        zvec[pl.ds(i * 16, 16)] = jnp.zeros((16,), jnp.float32)
        return carry

    lax.fori_loop(0, 200, zb, 0)

    pltpu.sync_copy(zvec.at[pl.ds(0, 3120)], s_sh.at[pl.ds(s * 3120, 3120)])

    @pl.when(s == 0)
    def _():
        pltpu.sync_copy(zvec.at[pl.ds(0, 80)], s_sh.at[pl.ds(16 * 3120, 80)])

    plsc.subcore_barrier()

    c0 = wid * P1_ITERS

    def valid(e):
        return (c0 + e) < NCHUNK

    def issue(e, b):
        hq = pltpu.async_copy(q_h.at[dstb.at[b]], qb.at[b], semr[b])
        hk = pltpu.async_copy(k_h.at[srcb.at[b]], kb.at[b], semr[b])
        return c0 + e, hq, hk

    def compute(e, b, st):
        base, hq, hk = st
        hq.wait()
        hk.wait()
        for g in range(CHUNK // 16):
            ev = jnp.zeros((16,), jnp.float32)
            for i in range(16):
                ei = g * 16 + i
                p = qb[b, ei, pl.ds(0, 16)] * kb[b, ei, pl.ds(0, 16)]
                for j in range(1, HID // 16):
                    p = p + qb[b, ei, pl.ds(j * 16, 16)] * kb[b, ei, pl.ds(j * 16, 16)]
                for sh in (8, 4, 2, 1):
                    p = p + _permute16(p, (lane + sh) & 15)
                ev = jnp.where(lane == i, p, ev)
            eb[b, pl.ds(g * 16, 16)] = jnp.exp(ev * 0.125)

        @pl.when(valid(e))
        def _():
            pltpu.sync_copy(eb.at[b], e_h.at[base])
            pltpu.sync_copy(eb.at[b], s_sh.at[dstb.at[b]], add=True)

    def loop(i, carry):
        pbase = c0 + 2 * i
        pltpu.sync_copy(dst_h.at[pl.ds(pbase, 2), :], dstb)
        pltpu.sync_copy(src_h.at[pl.ds(pbase, 2), :], srcb)
        sa = issue(2 * i, 0)
        sb = issue(2 * i + 1, 1)
        compute(2 * i, 0, sa)
        compute(2 * i + 1, 1, sb)
        return carry

    lax.fori_loop(0, P1_ITERS // 2, loop, 0)
    plsc.subcore_barrier()
    pltpu.sync_copy(s_sh.at[pl.ds(s * 3120, 3120)],
                    sp_h.at[c, pl.ds(s * 3120, 3120)])

    @pl.when(s == 0)
    def _():
        pltpu.sync_copy(s_sh.at[pl.ds(16 * 3120, 80)],
                        sp_h.at[c, pl.ds(16 * 3120, 80)])


def _make_edge_logits():
    return functools.partial(
        pl.kernel,
        out_type=[jax.ShapeDtypeStruct((E_PAD // CHUNK, CHUNK), jnp.float32),
                  jax.ShapeDtypeStruct((NC, N), jnp.float32)],
        mesh=_sc_mesh(),
        compiler_params=_sc_params,
        scratch_types=[
            pltpu.VMEM((2, CHUNK), jnp.int32),
            pltpu.VMEM((2, CHUNK), jnp.int32),
            pltpu.VMEM((2, CHUNK, HID), jnp.float32),
            pltpu.VMEM((2, CHUNK, HID), jnp.float32),
            pltpu.VMEM((2, CHUNK), jnp.float32),
            pltpu.VMEM((3200,), jnp.float32),
            pltpu.VMEM_SHARED((N,), jnp.float32),
            pltpu.SemaphoreType.DMA,
            pltpu.SemaphoreType.DMA,
        ],
    )(_edge_logits_body)


def _edge_agg_body(dst_h, src_h, e_h, v_h, out_h, dstb, srcb, eb, locb, vb,
                   zbuf, agg_sh, semv0, semv1):
    c = lax.axis_index("c")
    s = lax.axis_index("s")
    lane = lax.iota(jnp.int32, 16)
    lo = c * HALF
    semv = (semv0, semv1)

    def zb(i, carry):
        for j in range(HID // 16):
            zbuf[i, pl.ds(j * 16, 16)] = jnp.zeros((16,), jnp.float32)
        return carry

    lax.fori_loop(0, 64, zb, 0)

    def za(i, carry):
        pltpu.sync_copy(zbuf, agg_sh.at[pl.ds(s * 1568 + i * 64, 64), :])
        return carry

    lax.fori_loop(0, 24, za, 0)
    pltpu.sync_copy(zbuf.at[pl.ds(0, 32), :],
                    agg_sh.at[pl.ds(s * 1568 + 1536, 32), :])
    plsc.subcore_barrier()

    c0 = s * P2_ITERS

    def valid(e):
        return (c0 + e) < NCHUNK

    def issue(e, b):
        return pltpu.async_copy(v_h.at[srcb.at[b]], vb.at[b], semv[b])

    def compute(e, b, hv):
        hv.wait()
        for g in range(CHUNK // 16):
            dv = dstb[b, pl.ds(g * 16, 16)]
            ev = eb[b, pl.ds(g * 16, 16)]
            inm = (dv >= lo) & (dv < lo + HALF)
            em = jnp.where(inm, ev, 0.0)
            locb[b, pl.ds(g * 16, 16)] = jnp.where(inm, dv - lo, HALF)
            for i in range(16):
                ei = g * 16 + i
                emi = _permute16(em, jnp.full((16,), i, jnp.int32))
                for j in range(HID // 16):
                    vb[b, ei, pl.ds(j * 16, 16)] = vb[b, ei, pl.ds(j * 16, 16)] * emi

        @pl.when(valid(e))
        def _():
            pltpu.sync_copy(vb.at[b], agg_sh.at[locb.at[b]], add=True)

    def loop(i, carry):
        pbase = c0 + 2 * i
        pltpu.sync_copy(dst_h.at[pl.ds(pbase, 2), :], dstb)
        pltpu.sync_copy(src_h.at[pl.ds(pbase, 2), :], srcb)
        pltpu.sync_copy(e_h.at[pl.ds(pbase, 2), :], eb)
        ha = issue(2 * i, 0)
        hb = issue(2 * i + 1, 1)
        compute(2 * i, 0, ha)
        compute(2 * i + 1, 1, hb)
        return carry

    lax.fori_loop(0, P2_ITERS // 2, loop, 0)
    plsc.subcore_barrier()

    r0 = s * 1563

    @pl.when(s < NS - 1)
    def _():
        pltpu.sync_copy(agg_sh.at[pl.ds(r0, 1563), :],
                        out_h.at[pl.ds(lo + r0, 1563), :])

    @pl.when(s == NS - 1)
    def _():
        pltpu.sync_copy(agg_sh.at[pl.ds(r0, 1555), :],
                        out_h.at[pl.ds(lo + r0, 1555), :])


def _make_edge_agg():
    return functools.partial(
        pl.kernel,
        out_type=jax.ShapeDtypeStruct((N, HID), jnp.float32),
        mesh=_sc_mesh(),
        compiler_params=_sc_params,
        scratch_types=[
            pltpu.VMEM((2, CHUNK), jnp.int32),
            pltpu.VMEM((2, CHUNK), jnp.int32),
            pltpu.VMEM((2, CHUNK), jnp.float32),
            pltpu.VMEM((2, CHUNK), jnp.int32),
            pltpu.VMEM((2, CHUNK, HID), jnp.float32),
            pltpu.VMEM((64, HID), jnp.float32),
            pltpu.VMEM_SHARED((AGG_ROWS, HID), jnp.float32),
            pltpu.SemaphoreType.DMA,
            pltpu.SemaphoreType.DMA,
        ],
    )(_edge_agg_body)


def _t0_body(idx_r, ar_r, dr_r, gn_r, emb_r, Win_r, bin_r, Wq_r, bq_r,
             Wk_r, bk_r, Wv_r, bv_r, Ws_r, bs_r,
             q_r, k_r, v_r, base_r, xp_r):
    idx = jnp.reshape(idx_r[...], (BLK, 1))
    oh = (idx == lax.broadcasted_iota(jnp.int32, (BLK, 31), 1)).astype(jnp.float32)
    xe = jnp.dot(oh, emb_r[...], preferred_element_type=jnp.float32)
    ar = jnp.reshape(ar_r[...], (BLK, 1)).astype(jnp.float32)
    dr = jnp.reshape(dr_r[...], (BLK, 1)).astype(jnp.float32)
    gn = jnp.reshape(gn_r[...], (BLK, 1))
    x = jnp.concatenate([xe, ar, dr, gn], axis=1)

    def lin(W_r, b_r):
        return jnp.dot(x, W_r[...], preferred_element_type=jnp.float32) + b_r[...]

    xp_r[...] = lin(Win_r, bin_r)
    q_r[...] = lin(Wq_r, bq_r)
    k_r[...] = lin(Wk_r, bk_r)
    v_r[...] = lin(Wv_r, bv_r)
    base_r[...] = lin(Ws_r, bs_r)


def _post_layer(agg_r, sp_r, base_r, res_r, g_r, b_r):
    sv = jnp.sum(sp_r[...], axis=2).reshape(BLK, 1)
    sv = jnp.where(sv > 0.0, sv, 1.0)
    y = agg_r[...] / sv + base_r[...]
    mu = jnp.mean(y, axis=1, keepdims=True)
    var = jnp.mean((y - mu) ** 2, axis=1, keepdims=True)
    yn = (y - mu) / jnp.sqrt(var + 1e-5) * g_r[...] + b_r[...]
    return yn * jax.nn.sigmoid(yn) + res_r[...]


def _t1_body(agg_r, sp_r, base_r, xp_r, g_r, b_r, Wq_r, bq_r,
             Wk_r, bk_r, Wv_r, bv_r, Ws_r, bs_r,
             x1_r, q_r, k_r, v_r, base1_r):
    x1 = _post_layer(agg_r, sp_r, base_r, xp_r, g_r, b_r)
    x1_r[...] = x1

    def lin(W_r, b_r):
        return jnp.dot(x1, W_r[...], preferred_element_type=jnp.float32) + b_r[...]

    q_r[...] = lin(Wq_r, bq_r)
    k_r[...] = lin(Wk_r, bk_r)
    v_r[...] = lin(Wv_r, bv_r)
    base1_r[...] = lin(Ws_r, bs_r)


def _t2_body(agg_r, sp_r, base_r, x1_r, g_r, b_r, batch_r, bk_r, pc_r,
             Wbb_r, bbb_r, Wth_r, bth_r, Wrt_r, brt_r,
             Wa1_r, ba1_r, Wa2_r, ba2_r,
             th_r, rt_r, aux_r,
             acc_sum, acc_sq, acc_mx, acc_cnt):
    i = pl.program_id(0)

    @pl.when(i == 0)
    def _():
        acc_sum[...] = jnp.zeros((G, HID), jnp.float32)
        acc_sq[...] = jnp.zeros((G, HID), jnp.float32)
        acc_mx[...] = jnp.full((G, HID), -jnp.inf, jnp.float32)
        acc_cnt[...] = jnp.zeros((G, 1), jnp.float32)

    x2 = _post_layer(agg_r, sp_r, base_r, x1_r, g_r, b_r)
    bcol = jnp.reshape(batch_r[...], (BLK, 1))
    g0 = batch_r[0, 0, 0]
    g1 = batch_r[0, 0, BLK - 1]

    def graph_body(g, carry):
        m = bcol == g
        xm = jnp.where(m, x2, 0.0)
        acc_sum[pl.ds(g, 1), :] += jnp.sum(xm, axis=0, keepdims=True)
        acc_sq[pl.ds(g, 1), :] += jnp.sum(xm * xm, axis=0, keepdims=True)
        mxg = jnp.max(jnp.where(m, x2, -jnp.inf), axis=0, keepdims=True)
        acc_mx[pl.ds(g, 1), :] = jnp.maximum(acc_mx[pl.ds(g, 1), :], mxg)
        acc_cnt[pl.ds(g, 1), :] += jnp.reshape(
            jnp.sum(m.astype(jnp.float32)), (1, 1))
        return carry

    lax.fori_loop(g0, g1 + 1, graph_body, 0)

    @pl.when(i == NBLK - 1)
    def _():
        cnt = jnp.maximum(acc_cnt[...], 1.0)
        mean = acc_sum[...] / cnt
        msq = acc_sq[...] / cnt
        std = jnp.sqrt(jnp.clip(msq - mean * mean, 1e-6, None))
        mxv = acc_mx[...]
        mx = jnp.where(mxv == -jnp.inf, 0.0, mxv)
        gnn = jnp.concatenate([mean, mx, std], axis=1)
        comb = jnp.concatenate([gnn, bk_r[...], pc_r[...]], axis=1)
        final = comb @ Wbb_r[...] + bbb_r[...]
        final = final * jax.nn.sigmoid(final)
        th_r[...] = final @ Wth_r[...] + bth_r[...]
        rt_r[...] = final @ Wrt_r[...] + brt_r[...]
        a1 = gnn @ Wa1_r[...] + ba1_r[...]
        a1 = a1 * jax.nn.sigmoid(a1)
        aux_r[...] = a1 @ Wa2_r[...] + ba2_r[...]


def _row_spec(width=HID):
    return pl.BlockSpec((BLK, width), lambda i: (i, 0))


def _full_spec(shape):
    nd = len(shape)
    return pl.BlockSpec(shape, lambda i: (0,) * nd)


def _idx_spec():
    return pl.BlockSpec((1, 1, BLK), lambda i: (i, 0, 0))


def kernel(gate_type_idx, gate_arity, is_directional, gate_index_norm,
           edge_index, batch, backend_bit, precision_bit, emb, W_in, b_in,
           Wq0, bq0, Wk0, bk0, Wv0, bv0, Ws0, bs0, ln0_g, ln0_b,
           Wq1, bq1, Wk1, bk1, Wv1, bv1, Ws1, bs1, ln1_g, ln1_b,
           W_bb, b_bb, W_th, b_th, W_rt, b_rt, W_a1, b_a1, W_a2, b_a2):
    f32 = jnp.float32
    zpad = jnp.zeros((E_PAD - E,), edge_index.dtype)
    src = jnp.concatenate([edge_index[0], zpad]).reshape(E_PAD // CHUNK, CHUNK)
    dst = jnp.concatenate([edge_index[1], zpad]).reshape(E_PAD // CHUNK, CHUNK)
    idx3 = gate_type_idx.reshape(NBLK, 1, BLK)
    ar3 = gate_arity.reshape(NBLK, 1, BLK)
    dr3 = is_directional.reshape(NBLK, 1, BLK)
    gn3 = gate_index_norm.reshape(NBLK, 1, BLK)
    bt3 = batch.reshape(NBLK, 1, BLK)
    row = lambda b: b.reshape(1, -1)

    nodef = jax.ShapeDtypeStruct((N, HID), f32)
    q0, k0, v0, base0, xproj = pl.pallas_call(
        _t0_body,
        grid=(NBLK,),
        in_specs=[_idx_spec(), _idx_spec(), _idx_spec(), _idx_spec(),
                  _full_spec((31, 16)), _full_spec((19, HID)), _full_spec((1, HID)),
                  _full_spec((19, HID)), _full_spec((1, HID)),
                  _full_spec((19, HID)), _full_spec((1, HID)),
                  _full_spec((19, HID)), _full_spec((1, HID)),
                  _full_spec((19, HID)), _full_spec((1, HID))],
        out_specs=[_row_spec()] * 5,
        out_shape=[nodef] * 5,
    )(idx3, ar3, dr3, gn3, emb, W_in, row(b_in), Wq0, row(bq0),
      Wk0, row(bk0), Wv0, row(bv0), Ws0, row(bs0))

    e0, sp0 = _make_edge_logits()(dst, src, q0, k0)
    sp0 = sp0.T.reshape(NBLK, BLK, NC)
    agg0 = _make_edge_agg()(dst, src, e0, v0)

    x1, q1, k1, v1, base1 = pl.pallas_call(
        _t1_body,
        grid=(NBLK,),
        in_specs=[_row_spec(), pl.BlockSpec((1, BLK, NC), lambda i: (i, 0, 0)), _row_spec(), _row_spec(),
                  _full_spec((1, HID)), _full_spec((1, HID)),
                  _full_spec((HID, HID)), _full_spec((1, HID)),
                  _full_spec((HID, HID)), _full_spec((1, HID)),
                  _full_spec((HID, HID)), _full_spec((1, HID)),
                  _full_spec((HID, HID)), _full_spec((1, HID))],
        out_specs=[_row_spec()] * 5,
        out_shape=[nodef] * 5,
    )(agg0, sp0, base0, xproj, row(ln0_g), row(ln0_b),
      Wq1, row(bq1), Wk1, row(bk1), Wv1, row(bv1), Ws1, row(bs1))

    e1, sp1 = _make_edge_logits()(dst, src, q1, k1)
    sp1 = sp1.T.reshape(NBLK, BLK, NC)
    agg1 = _make_edge_agg()(dst, src, e1, v1)

    th, rt, aux = pl.pallas_call(
        _t2_body,
        grid=(NBLK,),
        in_specs=[_row_spec(), pl.BlockSpec((1, BLK, NC), lambda i: (i, 0, 0)), _row_spec(), _row_spec(),
                  _full_spec((1, HID)), _full_spec((1, HID)),
                  _idx_spec(),
                  _full_spec((G, 1)), _full_spec((G, 1)),
                  _full_spec((3 * HID + 2, HID)), _full_spec((1, HID)),
                  _full_spec((HID, 10)), _full_spec((1, 10)),
                  _full_spec((HID, 1)), _full_spec((1, 1)),
                  _full_spec((3 * HID, HID)), _full_spec((1, HID)),
                  _full_spec((HID, 32)), _full_spec((1, 32))],
        out_specs=[_full_spec((G, 10)), _full_spec((G, 1)), _full_spec((G, 32))],
        out_shape=[jax.ShapeDtypeStruct((G, 10), f32),
                   jax.ShapeDtypeStruct((G, 1), f32),
                   jax.ShapeDtypeStruct((G, 32), f32)],
        scratch_shapes=[pltpu.VMEM((G, HID), f32), pltpu.VMEM((G, HID), f32),
                        pltpu.VMEM((G, HID), f32), pltpu.VMEM((G, 1), f32)],
    )(agg1, sp1, base1, x1, row(ln1_g), row(ln1_b), bt3,
      backend_bit.reshape(G, 1), precision_bit.reshape(G, 1),
      W_bb, row(b_bb), W_th, row(b_th), W_rt, row(b_rt),
      W_a1, row(b_a1), W_a2, row(b_a2))

    return th, rt[:, 0], aux


# speedup vs baseline: 1.3372x; 1.3372x over previous
import functools

import jax
import jax.numpy as jnp
from jax import lax
from jax.experimental import pallas as pl
from jax.experimental.pallas import tpu as pltpu
from jax.experimental.pallas import tpu_sc as plsc

N = 50000
E = 800000
G = 64
HID = 64

NC = 2
NS = 16
NW = NC * NS

CHUNK = 128
NCHUNK = E // CHUNK
P1_ITERS = 196
P2_ITERS = 392
E_PAD = 6272 * CHUNK

HALF = N // 2
AGG_ROWS = 25088

BLK = 2000
NBLK = N // BLK

_sc_params = pltpu.CompilerParams(use_tc_tiling_on_sc=False)


def _sc_mesh():
    return plsc.VectorSubcoreMesh(core_axis_name="c", subcore_axis_name="s")


def _permute16(x, idx):
    dn = lax.GatherDimensionNumbers(
        offset_dims=(), collapsed_slice_dims=(0,), start_index_map=(0,))
    return lax.gather(x, idx[:, None], dn, slice_sizes=(1,),
                      mode=lax.GatherScatterMode.PROMISE_IN_BOUNDS)


def _edge_logits_body(dst_h, src_h, q_h, k_h, e_h, sp_h, dstb, srcb, qb, kb,
                      eb, zvec, s_sh, semr0, semr1, semr2, semr3, semo0, semo1):
    c = lax.axis_index("c")
    s = lax.axis_index("s")
    wid = s * NC + c
    lane = lax.iota(jnp.int32, 16)
    semr = (semr0, semr1, semr2, semr3)
    semo = (semo0, semo1)

    def zb(i, carry):
        zvec[pl.ds(i * 16, 16)] = jnp.zeros((16,), jnp.float32)
        return carry

    lax.fori_loop(0, 200, zb, 0)

    pltpu.sync_copy(zvec.at[pl.ds(0, 3120)], s_sh.at[pl.ds(s * 3120, 3120)])

    @pl.when(s == 0)
    def _():
        pltpu.sync_copy(zvec.at[pl.ds(0, 80)], s_sh.at[pl.ds(16 * 3120, 80)])

    plsc.subcore_barrier()

    c0 = wid * P1_ITERS

    def valid(e):
        return (c0 + e) < NCHUNK

    def gather(e, b):
        hq = pltpu.async_copy(q_h.at[dstb.at[b]], qb.at[b], semr[b])
        hk = pltpu.async_copy(k_h.at[srcb.at[b]], kb.at[b], semr[b])
        return hq, hk

    def compute(e, b, ob, st):
        hq, hk = st
        hq.wait()
        hk.wait()
        vf = jnp.where(valid(e), 1.0, 0.0).astype(jnp.float32)

        def grp(g, carry):
            ev = jnp.zeros((16,), jnp.float32)
            for i in range(16):
                ei = g * 16 + i
                p = qb[b, ei, pl.ds(0, 16)] * kb[b, ei, pl.ds(0, 16)]
                for j in range(1, HID // 16):
                    p = p + qb[b, ei, pl.ds(j * 16, 16)] * kb[b, ei, pl.ds(j * 16, 16)]
                for sh in (8, 4, 2, 1):
                    p = p + _permute16(p, (lane + sh) & 15)
                ev = jnp.where(lane == i, p, ev)
            eb[ob, pl.ds(g * 16, 16)] = jnp.exp(ev * 0.125) * vf
            return carry

        lax.fori_loop(0, CHUNK // 16, grp, 0)

    def outs(e, b, ob):
        pltpu.sync_copy(eb.at[ob], e_h.at[c0 + e])
        return pltpu.async_copy(eb.at[ob], s_sh.at[dstb.at[b]], semo[ob % 2], add=True)

    def wait_outs(hs):
        hs.wait()

    def loop(i, carry):
        e0 = 4 * i
        pltpu.sync_copy(dst_h.at[pl.ds(c0 + e0, 4), :], dstb)
        pltpu.sync_copy(src_h.at[pl.ds(c0 + e0, 4), :], srcb)
        g0 = gather(e0, 0)
        g1 = gather(e0 + 1, 1)
        compute(e0, 0, 0, g0)
        g2 = gather(e0 + 2, 2)
        o0 = outs(e0, 0, 0)
        compute(e0 + 1, 1, 1, g1)
        g3 = gather(e0 + 3, 3)
        o1 = outs(e0 + 1, 1, 1)
        wait_outs(o0)
        compute(e0 + 2, 2, 2, g2)
        o2 = outs(e0 + 2, 2, 2)
        wait_outs(o1)
        compute(e0 + 3, 3, 3, g3)
        o3 = outs(e0 + 3, 3, 3)
        wait_outs(o2)
        wait_outs(o3)
        return carry

    lax.fori_loop(0, P1_ITERS // 4, loop, 0)
    plsc.subcore_barrier()
    pltpu.sync_copy(s_sh.at[pl.ds(s * 3120, 3120)],
                    sp_h.at[c, pl.ds(s * 3120, 3120)])

    @pl.when(s == 0)
    def _():
        pltpu.sync_copy(s_sh.at[pl.ds(16 * 3120, 80)],
                        sp_h.at[c, pl.ds(16 * 3120, 80)])


def _make_edge_logits():
    return functools.partial(
        pl.kernel,
        out_type=[jax.ShapeDtypeStruct((E_PAD // CHUNK, CHUNK), jnp.float32),
                  jax.ShapeDtypeStruct((NC, N), jnp.float32)],
        mesh=_sc_mesh(),
        compiler_params=_sc_params,
        scratch_types=[
            pltpu.VMEM((4, CHUNK), jnp.int32),
            pltpu.VMEM((4, CHUNK), jnp.int32),
            pltpu.VMEM((4, CHUNK, HID), jnp.float32),
            pltpu.VMEM((4, CHUNK, HID), jnp.float32),
            pltpu.VMEM((4, CHUNK), jnp.float32),
            pltpu.VMEM((3200,), jnp.float32),
            pltpu.VMEM_SHARED((N,), jnp.float32),
            pltpu.SemaphoreType.DMA,
            pltpu.SemaphoreType.DMA,
            pltpu.SemaphoreType.DMA,
            pltpu.SemaphoreType.DMA,
            pltpu.SemaphoreType.DMA,
            pltpu.SemaphoreType.DMA,
        ],
    )(_edge_logits_body)


def _edge_agg_body(dst_h, src_h, e_h, v_h, out_h, dstb, srcb, eb, locb, vb,
                   zbuf, agg_sh, semv0, semv1):
    c = lax.axis_index("c")
    s = lax.axis_index("s")
    lane = lax.iota(jnp.int32, 16)
    lo = c * HALF
    semv = (semv0, semv1)

    def zb(i, carry):
        for j in range(HID // 16):
            zbuf[i, pl.ds(j * 16, 16)] = jnp.zeros((16,), jnp.float32)
        return carry

    lax.fori_loop(0, 64, zb, 0)

    def za(i, carry):
        pltpu.sync_copy(zbuf, agg_sh.at[pl.ds(s * 1568 + i * 64, 64), :])
        return carry

    lax.fori_loop(0, 24, za, 0)
    pltpu.sync_copy(zbuf.at[pl.ds(0, 32), :],
                    agg_sh.at[pl.ds(s * 1568 + 1536, 32), :])
    plsc.subcore_barrier()

    c0 = s * P2_ITERS

    def valid(e):
        return (c0 + e) < NCHUNK

    def issue(e, b):
        return pltpu.async_copy(v_h.at[srcb.at[b]], vb.at[b], semv[b])

    def compute(e, b, hv):
        hv.wait()
        for g in range(CHUNK // 16):
            dv = dstb[b, pl.ds(g * 16, 16)]
            ev = eb[b, pl.ds(g * 16, 16)]
            inm = (dv >= lo) & (dv < lo + HALF)
            em = jnp.where(inm, ev, 0.0)
            locb[b, pl.ds(g * 16, 16)] = jnp.where(inm, dv - lo, HALF)
            for i in range(16):
                ei = g * 16 + i
                emi = _permute16(em, jnp.full((16,), i, jnp.int32))
                for j in range(HID // 16):
                    vb[b, ei, pl.ds(j * 16, 16)] = vb[b, ei, pl.ds(j * 16, 16)] * emi

        @pl.when(valid(e))
        def _():
            pltpu.sync_copy(vb.at[b], agg_sh.at[locb.at[b]], add=True)

    def loop(i, carry):
        pbase = c0 + 2 * i
        pltpu.sync_copy(dst_h.at[pl.ds(pbase, 2), :], dstb)
        pltpu.sync_copy(src_h.at[pl.ds(pbase, 2), :], srcb)
        pltpu.sync_copy(e_h.at[pl.ds(pbase, 2), :], eb)
        ha = issue(2 * i, 0)
        hb = issue(2 * i + 1, 1)
        compute(2 * i, 0, ha)
        compute(2 * i + 1, 1, hb)
        return carry

    lax.fori_loop(0, P2_ITERS // 2, loop, 0)
    plsc.subcore_barrier()

    r0 = s * 1563

    @pl.when(s < NS - 1)
    def _():
        pltpu.sync_copy(agg_sh.at[pl.ds(r0, 1563), :],
                        out_h.at[pl.ds(lo + r0, 1563), :])

    @pl.when(s == NS - 1)
    def _():
        pltpu.sync_copy(agg_sh.at[pl.ds(r0, 1555), :],
                        out_h.at[pl.ds(lo + r0, 1555), :])


def _make_edge_agg():
    return functools.partial(
        pl.kernel,
        out_type=jax.ShapeDtypeStruct((N, HID), jnp.float32),
        mesh=_sc_mesh(),
        compiler_params=_sc_params,
        scratch_types=[
            pltpu.VMEM((2, CHUNK), jnp.int32),
            pltpu.VMEM((2, CHUNK), jnp.int32),
            pltpu.VMEM((2, CHUNK), jnp.float32),
            pltpu.VMEM((2, CHUNK), jnp.int32),
            pltpu.VMEM((2, CHUNK, HID), jnp.float32),
            pltpu.VMEM((64, HID), jnp.float32),
            pltpu.VMEM_SHARED((AGG_ROWS, HID), jnp.float32),
            pltpu.SemaphoreType.DMA,
            pltpu.SemaphoreType.DMA,
        ],
    )(_edge_agg_body)


def _t0_body(idx_r, ar_r, dr_r, gn_r, emb_r, Win_r, bin_r, Wq_r, bq_r,
             Wk_r, bk_r, Wv_r, bv_r, Ws_r, bs_r,
             q_r, k_r, v_r, base_r, xp_r):
    idx = jnp.reshape(idx_r[...], (BLK, 1))
    oh = (idx == lax.broadcasted_iota(jnp.int32, (BLK, 31), 1)).astype(jnp.float32)
    xe = jnp.dot(oh, emb_r[...], preferred_element_type=jnp.float32)
    ar = jnp.reshape(ar_r[...], (BLK, 1)).astype(jnp.float32)
    dr = jnp.reshape(dr_r[...], (BLK, 1)).astype(jnp.float32)
    gn = jnp.reshape(gn_r[...], (BLK, 1))
    x = jnp.concatenate([xe, ar, dr, gn], axis=1)

    def lin(W_r, b_r):
        return jnp.dot(x, W_r[...], preferred_element_type=jnp.float32) + b_r[...]

    xp_r[...] = lin(Win_r, bin_r)
    q_r[...] = lin(Wq_r, bq_r)
    k_r[...] = lin(Wk_r, bk_r)
    v_r[...] = lin(Wv_r, bv_r)
    base_r[...] = lin(Ws_r, bs_r)


def _post_layer(agg_r, sp_r, base_r, res_r, g_r, b_r):
    sv = jnp.sum(sp_r[...], axis=2).reshape(BLK, 1)
    sv = jnp.where(sv > 0.0, sv, 1.0)
    y = agg_r[...] / sv + base_r[...]
    mu = jnp.mean(y, axis=1, keepdims=True)
    var = jnp.mean((y - mu) ** 2, axis=1, keepdims=True)
    yn = (y - mu) / jnp.sqrt(var + 1e-5) * g_r[...] + b_r[...]
    return yn * jax.nn.sigmoid(yn) + res_r[...]


def _t1_body(agg_r, sp_r, base_r, xp_r, g_r, b_r, Wq_r, bq_r,
             Wk_r, bk_r, Wv_r, bv_r, Ws_r, bs_r,
             x1_r, q_r, k_r, v_r, base1_r):
    x1 = _post_layer(agg_r, sp_r, base_r, xp_r, g_r, b_r)
    x1_r[...] = x1

    def lin(W_r, b_r):
        return jnp.dot(x1, W_r[...], preferred_element_type=jnp.float32) + b_r[...]

    q_r[...] = lin(Wq_r, bq_r)
    k_r[...] = lin(Wk_r, bk_r)
    v_r[...] = lin(Wv_r, bv_r)
    base1_r[...] = lin(Ws_r, bs_r)


def _t2_body(agg_r, sp_r, base_r, x1_r, g_r, b_r, batch_r, bk_r, pc_r,
             Wbb_r, bbb_r, Wth_r, bth_r, Wrt_r, brt_r,
             Wa1_r, ba1_r, Wa2_r, ba2_r,
             th_r, rt_r, aux_r,
             acc_sum, acc_sq, acc_mx, acc_cnt):
    i = pl.program_id(0)

    @pl.when(i == 0)
    def _():
        acc_sum[...] = jnp.zeros((G, HID), jnp.float32)
        acc_sq[...] = jnp.zeros((G, HID), jnp.float32)
        acc_mx[...] = jnp.full((G, HID), -jnp.inf, jnp.float32)
        acc_cnt[...] = jnp.zeros((G, 1), jnp.float32)

    x2 = _post_layer(agg_r, sp_r, base_r, x1_r, g_r, b_r)
    bcol = jnp.reshape(batch_r[...], (BLK, 1))
    g0 = batch_r[0, 0, 0]
    g1 = batch_r[0, 0, BLK - 1]

    def graph_body(g, carry):
        m = bcol == g
        xm = jnp.where(m, x2, 0.0)
        acc_sum[pl.ds(g, 1), :] += jnp.sum(xm, axis=0, keepdims=True)
        acc_sq[pl.ds(g, 1), :] += jnp.sum(xm * xm, axis=0, keepdims=True)
        mxg = jnp.max(jnp.where(m, x2, -jnp.inf), axis=0, keepdims=True)
        acc_mx[pl.ds(g, 1), :] = jnp.maximum(acc_mx[pl.ds(g, 1), :], mxg)
        acc_cnt[pl.ds(g, 1), :] += jnp.reshape(
            jnp.sum(m.astype(jnp.float32)), (1, 1))
        return carry

    lax.fori_loop(g0, g1 + 1, graph_body, 0)

    @pl.when(i == NBLK - 1)
    def _():
        cnt = jnp.maximum(acc_cnt[...], 1.0)
        mean = acc_sum[...] / cnt
        msq = acc_sq[...] / cnt
        std = jnp.sqrt(jnp.clip(msq - mean * mean, 1e-6, None))
        mxv = acc_mx[...]
        mx = jnp.where(mxv == -jnp.inf, 0.0, mxv)
        gnn = jnp.concatenate([mean, mx, std], axis=1)
        comb = jnp.concatenate([gnn, bk_r[...], pc_r[...]], axis=1)
        final = comb @ Wbb_r[...] + bbb_r[...]
        final = final * jax.nn.sigmoid(final)
        th_r[...] = final @ Wth_r[...] + bth_r[...]
        rt_r[...] = final @ Wrt_r[...] + brt_r[...]
        a1 = gnn @ Wa1_r[...] + ba1_r[...]
        a1 = a1 * jax.nn.sigmoid(a1)
        aux_r[...] = a1 @ Wa2_r[...] + ba2_r[...]


def _row_spec(width=HID):
    return pl.BlockSpec((BLK, width), lambda i: (i, 0))


def _full_spec(shape):
    nd = len(shape)
    return pl.BlockSpec(shape, lambda i: (0,) * nd)


def _idx_spec():
    return pl.BlockSpec((1, 1, BLK), lambda i: (i, 0, 0))


def kernel(gate_type_idx, gate_arity, is_directional, gate_index_norm,
           edge_index, batch, backend_bit, precision_bit, emb, W_in, b_in,
           Wq0, bq0, Wk0, bk0, Wv0, bv0, Ws0, bs0, ln0_g, ln0_b,
           Wq1, bq1, Wk1, bk1, Wv1, bv1, Ws1, bs1, ln1_g, ln1_b,
           W_bb, b_bb, W_th, b_th, W_rt, b_rt, W_a1, b_a1, W_a2, b_a2):
    f32 = jnp.float32
    zpad = jnp.zeros((E_PAD - E,), edge_index.dtype)
    src = jnp.concatenate([edge_index[0], zpad]).reshape(E_PAD // CHUNK, CHUNK)
    dst = jnp.concatenate([edge_index[1], zpad]).reshape(E_PAD // CHUNK, CHUNK)
    idx3 = gate_type_idx.reshape(NBLK, 1, BLK)
    ar3 = gate_arity.reshape(NBLK, 1, BLK)
    dr3 = is_directional.reshape(NBLK, 1, BLK)
    gn3 = gate_index_norm.reshape(NBLK, 1, BLK)
    bt3 = batch.reshape(NBLK, 1, BLK)
    row = lambda b: b.reshape(1, -1)

    nodef = jax.ShapeDtypeStruct((N, HID), f32)
    q0, k0, v0, base0, xproj = pl.pallas_call(
        _t0_body,
        grid=(NBLK,),
        in_specs=[_idx_spec(), _idx_spec(), _idx_spec(), _idx_spec(),
                  _full_spec((31, 16)), _full_spec((19, HID)), _full_spec((1, HID)),
                  _full_spec((19, HID)), _full_spec((1, HID)),
                  _full_spec((19, HID)), _full_spec((1, HID)),
                  _full_spec((19, HID)), _full_spec((1, HID)),
                  _full_spec((19, HID)), _full_spec((1, HID))],
        out_specs=[_row_spec()] * 5,
        out_shape=[nodef] * 5,
    )(idx3, ar3, dr3, gn3, emb, W_in, row(b_in), Wq0, row(bq0),
      Wk0, row(bk0), Wv0, row(bv0), Ws0, row(bs0))

    e0, sp0 = _make_edge_logits()(dst, src, q0, k0)
    sp0 = sp0.T.reshape(NBLK, BLK, NC)
    agg0 = _make_edge_agg()(dst, src, e0, v0)

    x1, q1, k1, v1, base1 = pl.pallas_call(
        _t1_body,
        grid=(NBLK,),
        in_specs=[_row_spec(), pl.BlockSpec((1, BLK, NC), lambda i: (i, 0, 0)), _row_spec(), _row_spec(),
                  _full_spec((1, HID)), _full_spec((1, HID)),
                  _full_spec((HID, HID)), _full_spec((1, HID)),
                  _full_spec((HID, HID)), _full_spec((1, HID)),
                  _full_spec((HID, HID)), _full_spec((1, HID)),
                  _full_spec((HID, HID)), _full_spec((1, HID))],
        out_specs=[_row_spec()] * 5,
        out_shape=[nodef] * 5,
    )(agg0, sp0, base0, xproj, row(ln0_g), row(ln0_b),
      Wq1, row(bq1), Wk1, row(bk1), Wv1, row(bv1), Ws1, row(bs1))

    e1, sp1 = _make_edge_logits()(dst, src, q1, k1)
    sp1 = sp1.T.reshape(NBLK, BLK, NC)
    agg1 = _make_edge_agg()(dst, src, e1, v1)

    th, rt, aux = pl.pallas_call(
        _t2_body,
        grid=(NBLK,),
        in_specs=[_row_spec(), pl.BlockSpec((1, BLK, NC), lambda i: (i, 0, 0)), _row_spec(), _row_spec(),
                  _full_spec((1, HID)), _full_spec((1, HID)),
                  _idx_spec(),
                  _full_spec((G, 1)), _full_spec((G, 1)),
                  _full_spec((3 * HID + 2, HID)), _full_spec((1, HID)),
                  _full_spec((HID, 10)), _full_spec((1, 10)),
                  _full_spec((HID, 1)), _full_spec((1, 1)),
                  _full_spec((3 * HID, HID)), _full_spec((1, HID)),
                  _full_spec((HID, 32)), _full_spec((1, 32))],
        out_specs=[_full_spec((G, 10)), _full_spec((G, 1)), _full_spec((G, 32))],
        out_shape=[jax.ShapeDtypeStruct((G, 10), f32),
                   jax.ShapeDtypeStruct((G, 1), f32),
                   jax.ShapeDtypeStruct((G, 32), f32)],
        scratch_shapes=[pltpu.VMEM((G, HID), f32), pltpu.VMEM((G, HID), f32),
                        pltpu.VMEM((G, HID), f32), pltpu.VMEM((G, 1), f32)],
    )(agg1, sp1, base1, x1, row(ln1_g), row(ln1_b), bt3,
      backend_bit.reshape(G, 1), precision_bit.reshape(G, 1),
      W_bb, row(b_bb), W_th, row(b_th), W_rt, row(b_rt),
      W_a1, row(b_a1), W_a2, row(b_a2))

    return th, rt[:, 0], aux
